# Initial kernel scaffold; baseline (speedup 1.0000x reference)
import jax
import jax.numpy as jnp
from jax.experimental import pallas as pl


def kernel(w, embedding_weight):
    raise NotImplementedError("write your pallas kernel here")


# speedup vs baseline: 3.8565x; 3.8565x over previous
import functools

import jax
import jax.numpy as jnp
from jax import lax
from jax.experimental import pallas as pl
from jax.experimental.pallas import tpu as pltpu
from jax.experimental.pallas import tpu_sc as plsc

B = 4096
L = 201
D = 64
N = B * L
NC = 2
NS = 16
NW = NC * NS
NR = N // NW
CHUNK = 64
NCH = NR // CHUNK

_MESH = plsc.VectorSubcoreMesh(
    core_axis_name="c", subcore_axis_name="s", num_cores=NC, num_subcores=NS
)


@functools.partial(
    pl.kernel,
    out_type=jax.ShapeDtypeStruct((N, D), jnp.float32),
    mesh=_MESH,
    scratch_types=[
        pltpu.VMEM((NCH, CHUNK), jnp.int32),
        pltpu.VMEM((CHUNK, D), jnp.float32),
        pltpu.SemaphoreType.DMA,
    ],
    compiler_params=pltpu.CompilerParams(use_tc_tiling_on_sc=False),
)
def _gather(table_hbm, idx_hbm, out_hbm, idx_v, rows_v, gsem):
    wid = lax.axis_index("s") * NC + lax.axis_index("c")
    pltpu.sync_copy(idx_hbm.at[wid], idx_v)
    base = wid * NR

    @pl.loop(0, NCH)
    def _chunk(c):
        pltpu.async_copy(table_hbm.at[idx_v.at[c]], rows_v, gsem).wait()
        pltpu.sync_copy(rows_v, out_hbm.at[pl.ds(base + c * CHUNK, CHUNK)])


def kernel(w, embedding_weight):
    idx = w.reshape(NW, NCH, CHUNK)
    out = _gather(embedding_weight, idx)
    return out.reshape(B, L, D)


# speedup vs baseline: 5.4149x; 1.4041x over previous
import functools

import jax
import jax.numpy as jnp
from jax import lax
from jax.experimental import pallas as pl
from jax.experimental.pallas import tpu as pltpu
from jax.experimental.pallas import tpu_sc as plsc

B = 4096
L = 201
D = 64
N = B * L
NC = 2
NS = 16
NW = NC * NS
NR = N // NW
CHUNK = 128
NCH = NR // CHUNK
NBUF = 3

_MESH = plsc.VectorSubcoreMesh(
    core_axis_name="c", subcore_axis_name="s", num_cores=NC, num_subcores=NS
)


@functools.partial(
    pl.kernel,
    out_type=jax.ShapeDtypeStruct((N, D), jnp.float32),
    mesh=_MESH,
    scratch_types=[
        pltpu.VMEM((NCH, CHUNK), jnp.int32),
        pltpu.VMEM((NBUF, CHUNK, D), jnp.float32),
        pltpu.SemaphoreType.DMA((NBUF,)),
        pltpu.SemaphoreType.DMA((NBUF,)),
    ],
    compiler_params=pltpu.CompilerParams(use_tc_tiling_on_sc=False),
)
def _gather(table_hbm, idx_hbm, out_hbm, idx_v, rows_v, gsem, ssem):
    wid = lax.axis_index("s") * NC + lax.axis_index("c")
    pltpu.sync_copy(idx_hbm.at[wid], idx_v)
    base = wid * NR

    def gather_desc(c, b):
        return pltpu.make_async_copy(
            table_hbm.at[idx_v.at[c]], rows_v.at[b], gsem.at[b]
        )

    def store_desc(c, b):
        return pltpu.make_async_copy(
            rows_v.at[b], out_hbm.at[pl.ds(base + c * CHUNK, CHUNK)], ssem.at[b]
        )

    for b in range(NBUF):
        gather_desc(b, b).start()

    @pl.loop(0, NCH, step=NBUF)
    def _outer(j):
        for b in range(NBUF):
            c = j + b
            gather_desc(c, b).wait()
            store_desc(c, b).start()

            @pl.when(c + NBUF < NCH)
            def _refill():
                store_desc(c, b).wait()
                gather_desc(c + NBUF, b).start()

    for b in range(NBUF):
        store_desc(NCH - NBUF + b, b).wait()


def kernel(w, embedding_weight):
    idx = w.reshape(NW, NCH, CHUNK)
    out = _gather(embedding_weight, idx)
    return out.reshape(B, L, D)


# speedup vs baseline: 5.4369x; 1.0041x over previous
import functools

import jax
import jax.numpy as jnp
from jax import lax
from jax.experimental import pallas as pl
from jax.experimental.pallas import tpu as pltpu
from jax.experimental.pallas import tpu_sc as plsc

B = 4096
L = 201
D = 64
N = B * L
NC = 2
NS = 16
NW = NC * NS
NR = N // NW
CHUNK = 128
NCH = NR // CHUNK
NBUF = 6
NMAIN = (NCH // NBUF) * NBUF

_MESH = plsc.VectorSubcoreMesh(
    core_axis_name="c", subcore_axis_name="s", num_cores=NC, num_subcores=NS
)


@functools.partial(
    pl.kernel,
    out_type=jax.ShapeDtypeStruct((N, D), jnp.float32),
    mesh=_MESH,
    scratch_types=[
        pltpu.VMEM((NCH, CHUNK), jnp.int32),
        pltpu.VMEM((NBUF, CHUNK, D), jnp.float32),
        pltpu.SemaphoreType.DMA((NBUF,)),
        pltpu.SemaphoreType.DMA((NBUF,)),
    ],
    compiler_params=pltpu.CompilerParams(use_tc_tiling_on_sc=False),
)
def _gather(table_hbm, idx_hbm, out_hbm, idx_v, rows_v, gsem, ssem):
    wid = lax.axis_index("s") * NC + lax.axis_index("c")
    pltpu.sync_copy(idx_hbm.at[wid], idx_v)
    base = wid * NR

    def gather_desc(c, b):
        return pltpu.make_async_copy(
            table_hbm.at[idx_v.at[c]], rows_v.at[b], gsem.at[b]
        )

    def store_desc(c, b):
        return pltpu.make_async_copy(
            rows_v.at[b], out_hbm.at[pl.ds(base + c * CHUNK, CHUNK)], ssem.at[b]
        )

    for b in range(NBUF):
        gather_desc(b, b).start()

    @pl.loop(0, NMAIN, step=NBUF)
    def _outer(j):
        for b in range(NBUF):
            c = j + b
            gather_desc(c, b).wait()
            store_desc(c, b).start()

            @pl.when(c + NBUF < NCH)
            def _refill():
                store_desc(c, b).wait()
                gather_desc(c + NBUF, b).start()

    for r in range(NMAIN, NCH):
        b = r % NBUF
        gather_desc(r, b).wait()
        store_desc(r, b).start()

    for c in range(NCH - NBUF, NCH):
        store_desc(c, c % NBUF).wait()


def kernel(w, embedding_weight):
    idx = w.reshape(NW, NCH, CHUNK)
    out = _gather(embedding_weight, idx)
    return out.reshape(B, L, D)
